# baseline (speedup 1.0000x reference)
import jax
import jax.numpy as jnp
from jax import lax
from jax.experimental import pallas as pl
from jax.experimental.pallas import tpu as pltpu
from jax.experimental.pallas import tpu_sc as plsc

N_POOLED = 50000
N_UNPOOLED = 100000
D = 512
NC, NS, L = 2, 16, 16
NW = NC * NS
IDS_CHUNK = 2000
N_ID_CHUNKS = N_POOLED // IDS_CHUNK
VECS_PER_CHUNK = IDS_CHUNK // L
HALF = VECS_PER_CHUNK // 2
FLAG_VECS = 196
CAP = 3200
CAPP = CAP + L
IDS_STAGE = 1664
DATA_ROWS = 1568
DATA_TRIPS = DATA_ROWS // 32
CHUNK = 32
NBUF = 4
ZC = 64


def _splat(x, idx):
    return lax.gather(
        x, idx[:, None],
        dimension_numbers=lax.GatherDimensionNumbers(
            offset_dims=(), collapsed_slice_dims=(0,), start_index_map=(0,)),
        slice_sizes=(1,),
        mode=lax.GatherScatterMode.PROMISE_IN_BOUNDS)


def _body(pooled_hbm, ids_hbm, zeros_hbm, pad_hbm, out_hbm,
          idsA, idsB, srcbuf, dstbuf, zdstbuf, flagbuf, padbuf,
          zbuf, rowA, rowB, rowC, rowD,
          sem_z, sem_g, sem_s, sem_i):
    cid = lax.axis_index("c")
    sid = lax.axis_index("s")
    wid = sid * NC + cid
    base = wid * 3120 + jnp.minimum(wid, 20) * 8
    rangew = jnp.where(wid < 20, 3128, 3120)

    pltpu.sync_copy(zeros_hbm, zbuf)
    pltpu.sync_copy(pad_hbm, padbuf)

    zero16 = jnp.zeros((L,), jnp.int32)

    def clear_body(i, _):
        flagbuf[pl.ds(i * L, L)] = zero16
        return 0
    lax.fori_loop(0, FLAG_VECS, clear_body, 0)

    lane0 = jnp.zeros((L,), jnp.int32)
    one16 = jnp.ones((L,), jnp.int32)
    iota16 = lax.iota(jnp.int32, L)

    def scan_chunk(buf):
        def vec_body(i, _):
            v = buf[pl.ds(i * L, L)]
            m = (v >= base) & (v < base + rangew)
            plsc.store_scatter(flagbuf, [v - base], one16, mask=m)
            return 0

        lax.fori_loop(0, VECS_PER_CHUNK, vec_body, 0)

    def load_ids(cc, buf):
        pltpu.async_copy(ids_hbm.at[pl.ds(cc * IDS_CHUNK, IDS_CHUNK)], buf,
                         sem_i)

    def wait_ids(buf):
        pltpu.make_async_copy(ids_hbm.at[pl.ds(0, IDS_CHUNK)], buf,
                              sem_i).wait()

    load_ids(0, idsA)
    load_ids(1, idsB)

    def chunk_pair(p, _):
        wait_ids(idsA)
        scan_chunk(idsA)

        @pl.when(2 * p + 2 < N_ID_CHUNKS)
        def _():
            load_ids(2 * p + 2, idsA)
        wait_ids(idsB)
        scan_chunk(idsB)

        @pl.when(2 * p + 3 < N_ID_CHUNKS)
        def _():
            load_ids(2 * p + 3, idsB)
        return 0

    lax.fori_loop(0, N_ID_CHUNKS // 2, chunk_pair, 0)
    wait_ids(idsA)
    scan_chunk(idsA)

    base_i = wid * 1562 + jnp.minimum(wid, 16)
    base_lin = (base_i // 8) * 8
    astart = jnp.minimum((base_i // L) * L, N_POOLED - IDS_STAGE)
    off = base_lin - astart
    pltpu.sync_copy(ids_hbm.at[pl.ds(astart, IDS_STAGE)],
                    idsA.at[pl.ds(0, IDS_STAGE)])

    def fill_body(k, _):
        dv = plsc.load_gather(idsA, [off + k * L + iota16])
        dstbuf[pl.ds(k * L, L)] = dv
        return 0

    lax.fori_loop(0, DATA_ROWS // L, fill_body, 0)

    lane15 = jnp.full((L,), 15, jnp.int32)

    def comp_body(k, carry):
        nz_vec, zmax_vec = carry
        rel = lax.iota(jnp.int32, L) + k * L
        f = flagbuf[pl.ds(k * L, L)]
        mz = (f == 0) & (rel < rangew)
        absrow = base + rel
        pfz = plsc.cumsum(mz.astype(jnp.int32))
        plsc.store_scatter(zdstbuf, [nz_vec + pfz - 1], absrow, mask=mz)
        vals = jnp.where(mz, absrow, -1)
        zmax_vec = jnp.maximum(zmax_vec, _splat(plsc.cummax(vals), lane15))
        return nz_vec + _splat(pfz, lane15), zmax_vec

    nz_vec, zmax_vec = lax.fori_loop(
        0, FLAG_VECS, comp_body,
        (jnp.zeros((L,), jnp.int32), jnp.full((L,), -1, jnp.int32)))
    nz = jnp.max(nz_vec)
    for j in range(ZC // L):
        zpadpos = nz + lax.iota(jnp.int32, L) + j * L
        plsc.store_scatter(zdstbuf, [zpadpos], zmax_vec)

    trips = DATA_TRIPS
    tripsz = (nz + ZC - 1) // ZC
    bufs = [rowA, rowB, rowC, rowD]

    def issue_gather(t, buf):
        pltpu.async_copy(
            pooled_hbm.at[pl.ds(base_lin + t * CHUNK, CHUNK)], buf, sem_g)

    def wait_gather(buf):
        pltpu.make_async_copy(
            pooled_hbm.at[pl.ds(0, CHUNK)], buf, sem_g).wait()

    def wait_scatter(sem):
        pltpu.make_async_copy(
            rowA, out_hbm.at[dstbuf.at[pl.ds(0, CHUNK)]], sem).wait()

    def step(t, mine, ahead2):
        wait_gather(mine)
        pltpu.async_copy(
            mine, out_hbm.at[dstbuf.at[pl.ds(t * CHUNK, CHUNK)]], sem_s)

        @pl.when(t + 2 < trips)
        def _():
            @pl.when(t >= 2)
            def _():
                wait_scatter(sem_s)
            issue_gather(t + 2, ahead2)

    issue_gather(0, rowA)
    issue_gather(1, rowB)

    def pipe(t, _):
        @pl.when(t < tripsz)
        def _():
            pltpu.async_copy(
                zbuf, out_hbm.at[zdstbuf.at[pl.ds(t * ZC, ZC)]], sem_z)

        @pl.when(t < trips)
        def _():
            for r in range(NBUF):
                @pl.when(lax.rem(t, NBUF) == r)
                def _(r=r):
                    step(t, bufs[r], bufs[(r + 2) % NBUF])
        return 0

    lax.fori_loop(0, jnp.maximum(trips, tripsz), pipe, 0)

    lax.fori_loop(0, jnp.minimum(trips, NBUF),
                  lambda i, _: (wait_scatter(sem_s), 0)[1], 0)
    def wait_zero(i, _):
        pltpu.make_async_copy(
            zbuf, out_hbm.at[zdstbuf.at[pl.ds(0, ZC)]], sem_z).wait()
        return 0

    lax.fori_loop(0, tripsz, wait_zero, 0)


def kernel(pooled_energy, node_ids, n_unpooled):
    ids32 = node_ids.astype(jnp.int32)
    zeros_in = jnp.zeros((ZC, D), jnp.float32)
    pad_in = jnp.broadcast_to(ids32[0], (L,))
    call = pl.kernel(
        _body,
        out_type=jax.ShapeDtypeStruct((N_UNPOOLED, D), jnp.float32),
        mesh=plsc.VectorSubcoreMesh(core_axis_name="c", subcore_axis_name="s"),
        compiler_params=pltpu.CompilerParams(needs_layout_passes=False),
        scratch_types=[
            pltpu.VMEM((IDS_CHUNK,), jnp.int32),
            pltpu.VMEM((IDS_CHUNK,), jnp.int32),
            pltpu.VMEM((CAPP,), jnp.int32),
            pltpu.VMEM((CAPP,), jnp.int32),
            pltpu.VMEM((CAP,), jnp.int32),
            pltpu.VMEM((FLAG_VECS * L,), jnp.int32),
            pltpu.VMEM((L,), jnp.int32),
            pltpu.VMEM((ZC, D), jnp.float32),
            pltpu.VMEM((CHUNK, D), jnp.float32),
            pltpu.VMEM((CHUNK, D), jnp.float32),
            pltpu.VMEM((CHUNK, D), jnp.float32),
            pltpu.VMEM((CHUNK, D), jnp.float32),
            pltpu.SemaphoreType.DMA,
            pltpu.SemaphoreType.DMA,
            pltpu.SemaphoreType.DMA,
            pltpu.SemaphoreType.DMA,
        ],
    )
    return call(pooled_energy, ids32, zeros_in, pad_in)


# speedup vs baseline: 1.0466x; 1.0466x over previous
import jax
import jax.numpy as jnp
from jax import lax
from jax.experimental import pallas as pl
from jax.experimental.pallas import tpu as pltpu
from jax.experimental.pallas import tpu_sc as plsc

N_POOLED = 50000
N_UNPOOLED = 100000
D = 512
NC, NS, L = 2, 16, 16
NW = NC * NS
IDS_CHUNK = 2000
N_ID_CHUNKS = N_POOLED // IDS_CHUNK
VECS_PER_CHUNK = IDS_CHUNK // L
HALF = VECS_PER_CHUNK // 2
FLAG_VECS = 196
CAP = 3200
CAPP = CAP + L
IDS_STAGE = 1664
DATA_ROWS = 1568
DATA_TRIPS = DATA_ROWS // 32
CHUNK = 32
NBUF = 4


def _splat(x, idx):
    return lax.gather(
        x, idx[:, None],
        dimension_numbers=lax.GatherDimensionNumbers(
            offset_dims=(), collapsed_slice_dims=(0,), start_index_map=(0,)),
        slice_sizes=(1,),
        mode=lax.GatherScatterMode.PROMISE_IN_BOUNDS)


def _body(pooled_hbm, ids_hbm, zeros_hbm, pad_hbm, out_hbm,
          idsA, idsB, srcbuf, dstbuf, zdstbuf, flagbuf, padbuf,
          zbuf, rowA, rowB, rowC, rowD,
          sem_z, sem_g, sem_s, sem_i):
    cid = lax.axis_index("c")
    sid = lax.axis_index("s")
    wid = sid * NC + cid
    base = wid * 3120 + jnp.minimum(wid, 20) * 8
    rangew = jnp.where(wid < 20, 3128, 3120)

    pltpu.sync_copy(zeros_hbm, zbuf)
    pltpu.sync_copy(pad_hbm, padbuf)

    zero16 = jnp.zeros((L,), jnp.int32)

    def clear_body(i, _):
        flagbuf[pl.ds(i * L, L)] = zero16
        return 0
    lax.fori_loop(0, FLAG_VECS, clear_body, 0)

    lane0 = jnp.zeros((L,), jnp.int32)
    one16 = jnp.ones((L,), jnp.int32)
    iota16 = lax.iota(jnp.int32, L)

    def scan_chunk(buf):
        def vec_body(i, _):
            v = buf[pl.ds(i * L, L)]
            m = (v >= base) & (v < base + rangew)
            plsc.store_scatter(flagbuf, [v - base], one16, mask=m)
            return 0

        lax.fori_loop(0, VECS_PER_CHUNK, vec_body, 0)

    def load_ids(cc, buf):
        pltpu.async_copy(ids_hbm.at[pl.ds(cc * IDS_CHUNK, IDS_CHUNK)], buf,
                         sem_i)

    def wait_ids(buf):
        pltpu.make_async_copy(ids_hbm.at[pl.ds(0, IDS_CHUNK)], buf,
                              sem_i).wait()

    load_ids(0, idsA)
    load_ids(1, idsB)

    def chunk_pair(p, _):
        wait_ids(idsA)
        scan_chunk(idsA)

        @pl.when(2 * p + 2 < N_ID_CHUNKS)
        def _():
            load_ids(2 * p + 2, idsA)
        wait_ids(idsB)
        scan_chunk(idsB)

        @pl.when(2 * p + 3 < N_ID_CHUNKS)
        def _():
            load_ids(2 * p + 3, idsB)
        return 0

    lax.fori_loop(0, N_ID_CHUNKS // 2, chunk_pair, 0)
    wait_ids(idsA)
    scan_chunk(idsA)

    base_i = wid * 1562 + jnp.minimum(wid, 16)
    base_lin = (base_i // 8) * 8
    astart = jnp.minimum((base_i // L) * L, N_POOLED - IDS_STAGE)
    off = base_lin - astart
    pltpu.sync_copy(ids_hbm.at[pl.ds(astart, IDS_STAGE)],
                    idsA.at[pl.ds(0, IDS_STAGE)])

    def fill_body(k, _):
        dv = plsc.load_gather(idsA, [off + k * L + iota16])
        dstbuf[pl.ds(k * L, L)] = dv
        return 0

    lax.fori_loop(0, DATA_ROWS // L, fill_body, 0)

    lane15 = jnp.full((L,), 15, jnp.int32)

    def comp_body(k, carry):
        nz_vec, zmax_vec = carry
        rel = lax.iota(jnp.int32, L) + k * L
        f = flagbuf[pl.ds(k * L, L)]
        mz = (f == 0) & (rel < rangew)
        absrow = base + rel
        pfz = plsc.cumsum(mz.astype(jnp.int32))
        plsc.store_scatter(zdstbuf, [nz_vec + pfz - 1], absrow, mask=mz)
        vals = jnp.where(mz, absrow, -1)
        zmax_vec = jnp.maximum(zmax_vec, _splat(plsc.cummax(vals), lane15))
        return nz_vec + _splat(pfz, lane15), zmax_vec

    nz_vec, zmax_vec = lax.fori_loop(
        0, FLAG_VECS, comp_body,
        (jnp.zeros((L,), jnp.int32), jnp.full((L,), -1, jnp.int32)))
    nz = jnp.max(nz_vec)
    for j in range(CHUNK // L):
        zpadpos = nz + lax.iota(jnp.int32, L) + j * L
        plsc.store_scatter(zdstbuf, [zpadpos], zmax_vec)

    trips = DATA_TRIPS
    tripsz = (nz + CHUNK - 1) // CHUNK
    bufs = [rowA, rowB, rowC, rowD]

    def issue_gather(t, buf):
        pltpu.async_copy(
            pooled_hbm.at[pl.ds(base_lin + t * CHUNK, CHUNK)], buf, sem_g)

    def wait_gather(buf):
        pltpu.make_async_copy(
            pooled_hbm.at[pl.ds(0, CHUNK)], buf, sem_g).wait()

    def wait_scatter(sem):
        pltpu.make_async_copy(
            rowA, out_hbm.at[dstbuf.at[pl.ds(0, CHUNK)]], sem).wait()

    def step(t, mine, ahead2):
        wait_gather(mine)
        pltpu.async_copy(
            mine, out_hbm.at[dstbuf.at[pl.ds(t * CHUNK, CHUNK)]], sem_s)

        @pl.when(t + 2 < trips)
        def _():
            @pl.when(t >= 2)
            def _():
                wait_scatter(sem_s)
            issue_gather(t + 2, ahead2)

    issue_gather(0, rowA)
    issue_gather(1, rowB)

    def pipe(t, _):
        @pl.when(t < tripsz)
        def _():
            pltpu.async_copy(
                zbuf, out_hbm.at[zdstbuf.at[pl.ds(t * CHUNK, CHUNK)]], sem_z)

        @pl.when(t < trips)
        def _():
            for r in range(NBUF):
                @pl.when(lax.rem(t, NBUF) == r)
                def _(r=r):
                    step(t, bufs[r], bufs[(r + 2) % NBUF])
        return 0

    lax.fori_loop(0, jnp.maximum(trips, tripsz), pipe, 0)

    lax.fori_loop(0, jnp.minimum(trips, NBUF),
                  lambda i, _: (wait_scatter(sem_s), 0)[1], 0)
    lax.fori_loop(0, tripsz,
                  lambda i, _: (wait_scatter(sem_z), 0)[1], 0)


def kernel(pooled_energy, node_ids, n_unpooled):
    ids32 = node_ids.astype(jnp.int32)
    zeros_in = jnp.zeros((CHUNK, D), jnp.float32)
    pad_in = jnp.broadcast_to(ids32[0], (L,))
    call = pl.kernel(
        _body,
        out_type=jax.ShapeDtypeStruct((N_UNPOOLED, D), jnp.float32),
        mesh=plsc.VectorSubcoreMesh(core_axis_name="c", subcore_axis_name="s"),
        compiler_params=pltpu.CompilerParams(needs_layout_passes=False),
        scratch_types=[
            pltpu.VMEM((IDS_CHUNK,), jnp.int32),
            pltpu.VMEM((IDS_CHUNK,), jnp.int32),
            pltpu.VMEM((CAPP,), jnp.int32),
            pltpu.VMEM((CAPP,), jnp.int32),
            pltpu.VMEM((CAP,), jnp.int32),
            pltpu.VMEM((FLAG_VECS * L,), jnp.int32),
            pltpu.VMEM((L,), jnp.int32),
            pltpu.VMEM((CHUNK, D), jnp.float32),
            pltpu.VMEM((CHUNK, D), jnp.float32),
            pltpu.VMEM((CHUNK, D), jnp.float32),
            pltpu.VMEM((CHUNK, D), jnp.float32),
            pltpu.VMEM((CHUNK, D), jnp.float32),
            pltpu.SemaphoreType.DMA,
            pltpu.SemaphoreType.DMA,
            pltpu.SemaphoreType.DMA,
            pltpu.SemaphoreType.DMA,
        ],
    )
    return call(pooled_energy, ids32, zeros_in, pad_in)
